# baseline (speedup 1.0000x reference)
import functools

import jax
import jax.numpy as jnp
from jax import lax
from jax.experimental import pallas as pl
from jax.experimental.pallas import tpu as pltpu
from jax.experimental.pallas import tpu_sc as plsc

V, D, L = 100000, 64, 2
B, S = 4096, 200

NC, NS, LANES = 2, 16, 16
NW = NC * NS
DEAD = V
VP = 102400
CBLK = 51200
SP = 208
RPT = B // NW
PBLK = 512
NSLAB = RPT // LANES


def _prep_body(ids_ref, msk_ref, sela_ref, selb_ref, inv_ref):
    ids = ids_ref[...]
    msk = msk_ref[...]
    sel = jnp.where(msk > 0, ids, DEAD)
    selp = jnp.concatenate(
        [sel, jnp.full((256 - S, PBLK), DEAD, jnp.int32)], axis=0)
    sela_ref[...] = selp[:128].T
    selb_ref[...] = selp[128:].T
    cnt = jnp.sum(msk.astype(jnp.float32), axis=0)
    inv = 1.0 / jnp.maximum(cnt, 1.0)
    inv_ref[...] = jnp.broadcast_to(inv[None, :], (8, PBLK))


_prep = pl.pallas_call(
    _prep_body,
    grid=(B // PBLK,),
    in_specs=[
        pl.BlockSpec((S, PBLK), lambda i: (0, i)),
        pl.BlockSpec((S, PBLK), lambda i: (0, i)),
    ],
    out_specs=[
        pl.BlockSpec((PBLK, 128), lambda i: (i, 0)),
        pl.BlockSpec((PBLK, 128), lambda i: (i, 0)),
        pl.BlockSpec((8, PBLK), lambda i: (0, i)),
    ],
    out_shape=[
        jax.ShapeDtypeStruct((B, 128), jnp.int32),
        jax.ShapeDtypeStruct((B, 128), jnp.int32),
        jax.ShapeDtypeStruct((8, B), jnp.float32),
    ],
)


def _proj_body(w_ref, embt_ref, out_ref):
    i = pl.program_id(0)
    y = lax.dot_general(w_ref[...], embt_ref[...],
                        (((1,), (0,)), ((), ())),
                        preferred_element_type=jnp.float32)
    cols = i * CBLK + lax.broadcasted_iota(jnp.int32, (L, CBLK), 1)
    y = jnp.where(cols < V, y, 0.0)
    yb = y.astype(jnp.bfloat16)
    lo = lax.bitcast_convert_type(yb[0, :], jnp.uint16).astype(jnp.uint32)
    hi = lax.bitcast_convert_type(yb[1, :], jnp.uint16).astype(jnp.uint32)
    out_ref[...] = lax.bitcast_convert_type(lo | (hi << 16), jnp.int32)


_proj_table = pl.pallas_call(
    _proj_body,
    grid=(VP // CBLK,),
    in_specs=[
        pl.BlockSpec((L, D), lambda i: (0, 0)),
        pl.BlockSpec((D, CBLK), lambda i: (0, i)),
    ],
    out_specs=pl.BlockSpec((CBLK,), lambda i: (i,)),
    out_shape=jax.ShapeDtypeStruct((VP,), jnp.int32),
)


def _rot(x, idx):
    return lax.gather(
        x, idx[:, None],
        lax.GatherDimensionNumbers(
            offset_dims=(), collapsed_slice_dims=(0,), start_index_map=(0,)),
        (1,), mode=lax.GatherScatterMode.PROMISE_IN_BOUNDS)


def _sc_body(tbl_hbm, sela_hbm, selb_hbm, inv_hbm, bias_hbm, out_hbm,
             tbl_v, sel_v, inv_v, bias_v, out_v, sems):
    wid = lax.axis_index("s") * NC + lax.axis_index("c")
    row0 = wid * RPT

    pltpu.sync_copy(tbl_hbm, tbl_v)
    pltpu.sync_copy(inv_hbm.at[0, pl.ds(row0, RPT)], inv_v)
    pltpu.sync_copy(bias_hbm, bias_v)

    def issue(slab, buf):
        rows = pl.ds(row0 + slab * LANES, LANES)
        pltpu.make_async_copy(
            sela_hbm.at[rows], sel_v.at[buf, 0], sems[buf]).start()
        pltpu.make_async_copy(
            selb_hbm.at[rows], sel_v.at[buf, 1], sems[buf]).start()

    def drain(buf):
        rows = pl.ds(row0, LANES)
        pltpu.make_async_copy(
            sela_hbm.at[rows], sel_v.at[buf, 0], sems[buf]).wait()
        pltpu.make_async_copy(
            selb_hbm.at[rows], sel_v.at[buf, 1], sems[buf]).wait()

    issue(0, 0)
    issue(1, 1)

    lane = lax.iota(jnp.int32, LANES)
    rot8 = (lane + 8) & 15
    rot4 = (lane + 4) & 15
    rot2 = (lane + 2) & 15
    rot1 = (lane + 1) & 15
    zero = jnp.zeros((LANES,), jnp.float32)
    zero32 = jnp.zeros((2 * LANES,), jnp.bfloat16)
    bias0 = _rot(bias_v[...], jnp.zeros((LANES,), jnp.int32))
    bias1 = _rot(bias_v[...], jnp.ones((LANES,), jnp.int32))

    def tree(x):
        x = x + _rot(x, rot8)
        x = x + _rot(x, rot4)
        x = x + _rot(x, rot2)
        return x + _rot(x, rot1)

    def slab_group(g, carry):
        for buf in range(2):
            slab = g * 2 + buf
            drain(buf)

            def row_body(r2, regs):
                sr0, sr1 = regs
                for k in range(2):
                    r = r2 * 2 + k
                    a = [zero32, zero32, zero32, zero32]
                    for c in range(SP // LANES):
                        half, cc = (0, c) if c < 8 else (1, c - 8)
                        sel16 = sel_v[buf, half, r, pl.ds(cc * LANES, LANES)]
                        g16 = plsc.load_gather(tbl_v, [sel16])
                        a[c % 4] = a[c % 4] + plsc.bitcast(g16, jnp.bfloat16)
                    ups = [plsc.unpack(x, format=plsc.PackFormat.INTERLEAVED)
                           for x in a]
                    t0 = tree((ups[0][0] + ups[1][0]) + (ups[2][0] + ups[3][0]))
                    t1 = tree((ups[0][1] + ups[1][1]) + (ups[2][1] + ups[3][1]))
                    sr0 = jnp.where(lane == r, t0, sr0)
                    sr1 = jnp.where(lane == r, t1, sr1)
                return (sr0, sr1)
            sr0, sr1 = lax.fori_loop(0, LANES // 2, row_body, (zero, zero))

            @pl.when(slab + 2 < NSLAB)
            def _():
                issue(slab + 2, buf)

            sl = pl.ds(slab * LANES, LANES)
            iv = inv_v[sl]
            out_v[0, sl] = sr0 * iv + bias0
            out_v[1, sl] = sr1 * iv + bias1
        return carry
    lax.fori_loop(0, NSLAB // 2, slab_group, 0)

    pltpu.sync_copy(out_v.at[0], out_hbm.at[0, pl.ds(row0, RPT)])
    pltpu.sync_copy(out_v.at[1], out_hbm.at[1, pl.ds(row0, RPT)])


_sc_logits = functools.partial(
    pl.kernel,
    out_type=jax.ShapeDtypeStruct((L, B), jnp.float32),
    mesh=plsc.VectorSubcoreMesh(core_axis_name="c", subcore_axis_name="s"),
    compiler_params=pltpu.CompilerParams(
        use_tc_tiling_on_sc=False, needs_layout_passes=False),
    scratch_types=[
        pltpu.VMEM((VP,), jnp.int32),
        pltpu.VMEM((2, 2, LANES, 128), jnp.int32),
        pltpu.VMEM((RPT,), jnp.float32),
        pltpu.VMEM((LANES,), jnp.float32),
        pltpu.VMEM((L, RPT), jnp.float32),
        [pltpu.SemaphoreType.DMA] * 2,
    ],
)(_sc_body)


def kernel(input_ids, attention_mask, emb_weight, fc_w, fc_b):
    sela, selb, inv8 = _prep(input_ids.T.astype(jnp.int32),
                             attention_mask.T.astype(jnp.int32))
    tbl = _proj_table(fc_w.astype(jnp.float32), emb_weight.T)
    bias16 = jnp.zeros((LANES,), jnp.float32).at[:L].set(fc_b)
    out2 = _sc_logits(tbl, sela, selb, inv8, bias16)
    return out2.T


# speedup vs baseline: 1.1790x; 1.1790x over previous
import functools

import jax
import jax.numpy as jnp
from jax import lax
from jax.experimental import pallas as pl
from jax.experimental.pallas import tpu as pltpu
from jax.experimental.pallas import tpu_sc as plsc

V, D, L = 100000, 64, 2
B, S = 4096, 200

NC, NS, LANES = 2, 16, 16
NW = NC * NS
DEAD = V
VP = 102400
CBLK = 51200
SP = 208
RPT = B // NW
PBLK = 512
NSLAB = RPT // LANES


def _prep_body(ids_ref, msk_ref, sela_ref, selb_ref, inv_ref):
    ids = ids_ref[...]
    msk = msk_ref[...]
    sel = jnp.where(msk > 0, ids, DEAD)
    selp = jnp.concatenate(
        [sel, jnp.full((256 - S, PBLK), DEAD, jnp.int32)], axis=0)
    sela_ref[...] = selp[:128].T
    selb_ref[...] = selp[128:].T
    cnt = jnp.sum(msk.astype(jnp.float32), axis=0)
    inv = 1.0 / jnp.maximum(cnt, 1.0)
    inv_ref[...] = jnp.broadcast_to(inv[None, :], (8, PBLK))


_prep = pl.pallas_call(
    _prep_body,
    grid=(B // PBLK,),
    in_specs=[
        pl.BlockSpec((S, PBLK), lambda i: (0, i)),
        pl.BlockSpec((S, PBLK), lambda i: (0, i)),
    ],
    out_specs=[
        pl.BlockSpec((PBLK, 128), lambda i: (i, 0)),
        pl.BlockSpec((PBLK, 128), lambda i: (i, 0)),
        pl.BlockSpec((8, PBLK), lambda i: (0, i)),
    ],
    out_shape=[
        jax.ShapeDtypeStruct((B, 128), jnp.int32),
        jax.ShapeDtypeStruct((B, 128), jnp.int32),
        jax.ShapeDtypeStruct((8, B), jnp.float32),
    ],
)


def _proj_body(w_ref, embt_ref, out_ref):
    i = pl.program_id(0)
    y = lax.dot_general(w_ref[...], embt_ref[...],
                        (((1,), (0,)), ((), ())),
                        preferred_element_type=jnp.float32)
    cols = i * CBLK + lax.broadcasted_iota(jnp.int32, (L, CBLK), 1)
    y = jnp.where(cols < V, y, 0.0)
    yb = y.astype(jnp.bfloat16)
    lo = lax.bitcast_convert_type(yb[0, :], jnp.uint16).astype(jnp.uint32)
    hi = lax.bitcast_convert_type(yb[1, :], jnp.uint16).astype(jnp.uint32)
    out_ref[...] = lax.bitcast_convert_type(lo | (hi << 16), jnp.int32)


_proj_table = pl.pallas_call(
    _proj_body,
    grid=(VP // CBLK,),
    in_specs=[
        pl.BlockSpec((L, D), lambda i: (0, 0)),
        pl.BlockSpec((D, CBLK), lambda i: (0, i)),
    ],
    out_specs=pl.BlockSpec((CBLK,), lambda i: (i,)),
    out_shape=jax.ShapeDtypeStruct((VP,), jnp.int32),
)


def _rot(x, idx):
    return lax.gather(
        x, idx[:, None],
        lax.GatherDimensionNumbers(
            offset_dims=(), collapsed_slice_dims=(0,), start_index_map=(0,)),
        (1,), mode=lax.GatherScatterMode.PROMISE_IN_BOUNDS)


def _sc_body(tbl_hbm, sela_hbm, selb_hbm, inv_hbm, bias_hbm, out_hbm,
             tbl_v, sel_v, inv_v, bias_v, out_v, tbl_sp, sems):
    sid = lax.axis_index("s")
    wid = sid * NC + lax.axis_index("c")
    row0 = wid * RPT

    def issue(slab, buf):
        rows = pl.ds(row0 + slab * LANES, LANES)
        pltpu.make_async_copy(
            sela_hbm.at[rows], sel_v.at[buf, 0], sems[buf]).start()
        pltpu.make_async_copy(
            selb_hbm.at[rows], sel_v.at[buf, 1], sems[buf]).start()

    def drain(buf):
        rows = pl.ds(row0, LANES)
        pltpu.make_async_copy(
            sela_hbm.at[rows], sel_v.at[buf, 0], sems[buf]).wait()
        pltpu.make_async_copy(
            selb_hbm.at[rows], sel_v.at[buf, 1], sems[buf]).wait()

    issue(0, 0)
    issue(1, 1)
    pltpu.make_async_copy(
        inv_hbm.at[0, pl.ds(row0, RPT)], inv_v, sems[2]).start()
    pltpu.make_async_copy(bias_hbm, bias_v, sems[2]).start()

    @pl.when(sid == 0)
    def _():
        pltpu.sync_copy(tbl_hbm, tbl_sp)
    plsc.subcore_barrier()
    pltpu.sync_copy(tbl_sp, tbl_v)

    pltpu.make_async_copy(
        inv_hbm.at[0, pl.ds(row0, RPT)], inv_v, sems[2]).wait()
    pltpu.make_async_copy(bias_hbm, bias_v, sems[2]).wait()

    lane = lax.iota(jnp.int32, LANES)
    rot8 = (lane + 8) & 15
    rot4 = (lane + 4) & 15
    rot2 = (lane + 2) & 15
    rot1 = (lane + 1) & 15
    zero = jnp.zeros((LANES,), jnp.float32)
    zero32 = jnp.zeros((2 * LANES,), jnp.bfloat16)
    bias0 = _rot(bias_v[...], jnp.zeros((LANES,), jnp.int32))
    bias1 = _rot(bias_v[...], jnp.ones((LANES,), jnp.int32))

    def tree(x):
        x = x + _rot(x, rot8)
        x = x + _rot(x, rot4)
        x = x + _rot(x, rot2)
        return x + _rot(x, rot1)

    def slab_group(g, carry):
        for buf in range(2):
            slab = g * 2 + buf
            drain(buf)

            def row_body(r2, regs):
                sr0, sr1 = regs
                for k in range(2):
                    r = r2 * 2 + k
                    a = [zero32, zero32, zero32, zero32]
                    for c in range(SP // LANES):
                        half, cc = (0, c) if c < 8 else (1, c - 8)
                        sel16 = sel_v[buf, half, r, pl.ds(cc * LANES, LANES)]
                        g16 = plsc.load_gather(tbl_v, [sel16])
                        a[c % 4] = a[c % 4] + plsc.bitcast(g16, jnp.bfloat16)
                    ups = [plsc.unpack(x, format=plsc.PackFormat.INTERLEAVED)
                           for x in a]
                    t0 = tree((ups[0][0] + ups[1][0]) + (ups[2][0] + ups[3][0]))
                    t1 = tree((ups[0][1] + ups[1][1]) + (ups[2][1] + ups[3][1]))
                    sr0 = jnp.where(lane == r, t0, sr0)
                    sr1 = jnp.where(lane == r, t1, sr1)
                return (sr0, sr1)
            sr0, sr1 = lax.fori_loop(0, LANES // 2, row_body, (zero, zero))

            @pl.when(slab + 2 < NSLAB)
            def _():
                issue(slab + 2, buf)

            sl = pl.ds(slab * LANES, LANES)
            iv = inv_v[sl]
            out_v[0, sl] = sr0 * iv + bias0
            out_v[1, sl] = sr1 * iv + bias1
        return carry
    lax.fori_loop(0, NSLAB // 2, slab_group, 0)

    pltpu.sync_copy(out_v.at[0], out_hbm.at[0, pl.ds(row0, RPT)])
    pltpu.sync_copy(out_v.at[1], out_hbm.at[1, pl.ds(row0, RPT)])


_sc_logits = functools.partial(
    pl.kernel,
    out_type=jax.ShapeDtypeStruct((L, B), jnp.float32),
    mesh=plsc.VectorSubcoreMesh(core_axis_name="c", subcore_axis_name="s"),
    compiler_params=pltpu.CompilerParams(
        use_tc_tiling_on_sc=False, needs_layout_passes=False),
    scratch_types=[
        pltpu.VMEM((VP,), jnp.int32),
        pltpu.VMEM((2, 2, LANES, 128), jnp.int32),
        pltpu.VMEM((RPT,), jnp.float32),
        pltpu.VMEM((LANES,), jnp.float32),
        pltpu.VMEM((L, RPT), jnp.float32),
        pltpu.VMEM_SHARED((VP,), jnp.int32),
        [pltpu.SemaphoreType.DMA] * 3,
    ],
)(_sc_body)


def kernel(input_ids, attention_mask, emb_weight, fc_w, fc_b):
    sela, selb, inv8 = _prep(input_ids.T.astype(jnp.int32),
                             attention_mask.T.astype(jnp.int32))
    tbl = _proj_table(fc_w.astype(jnp.float32), emb_weight.T)
    bias16 = jnp.zeros((LANES,), jnp.float32).at[:L].set(fc_b)
    out2 = _sc_logits(tbl, sela, selb, inv8, bias16)
    return out2.T


# speedup vs baseline: 1.1817x; 1.0023x over previous
import functools

import jax
import jax.numpy as jnp
from jax import lax
from jax.experimental import pallas as pl
from jax.experimental.pallas import tpu as pltpu
from jax.experimental.pallas import tpu_sc as plsc

V, D, L = 100000, 64, 2
B, S = 4096, 200

NC, NS, LANES = 2, 16, 16
NW = NC * NS
DEAD = V
VP = 102400
CBLK = 51200
SP = 208
RPT = B // NW
PBLK = 512
NSLAB = RPT // LANES


def _prep_body(ids_ref, msk_ref, sela_ref, selb_ref, inv_ref):
    ids = ids_ref[...]
    msk = msk_ref[...]
    sel = jnp.where(msk > 0, ids, DEAD)
    selp = jnp.concatenate(
        [sel, jnp.full((256 - S, PBLK), DEAD, jnp.int32)], axis=0)
    sela_ref[...] = selp[:128].T
    selb_ref[...] = selp[128:].T
    cnt = jnp.sum(msk.astype(jnp.float32), axis=0)
    inv = 1.0 / jnp.maximum(cnt, 1.0)
    inv_ref[...] = jnp.broadcast_to(inv[None, :], (8, PBLK))


_prep = pl.pallas_call(
    _prep_body,
    grid=(B // PBLK,),
    in_specs=[
        pl.BlockSpec((S, PBLK), lambda i: (0, i)),
        pl.BlockSpec((S, PBLK), lambda i: (0, i)),
    ],
    out_specs=[
        pl.BlockSpec((PBLK, 128), lambda i: (i, 0)),
        pl.BlockSpec((PBLK, 128), lambda i: (i, 0)),
        pl.BlockSpec((8, PBLK), lambda i: (0, i)),
    ],
    out_shape=[
        jax.ShapeDtypeStruct((B, 128), jnp.int32),
        jax.ShapeDtypeStruct((B, 128), jnp.int32),
        jax.ShapeDtypeStruct((8, B), jnp.float32),
    ],
)


def _proj_body(w_ref, embt_ref, out_ref):
    i = pl.program_id(0)
    y = lax.dot_general(w_ref[...], embt_ref[...],
                        (((1,), (0,)), ((), ())),
                        preferred_element_type=jnp.float32)
    cols = i * CBLK + lax.broadcasted_iota(jnp.int32, (L, CBLK), 1)
    y = jnp.where(cols < V, y, 0.0)
    yb = y.astype(jnp.bfloat16)
    lo = lax.bitcast_convert_type(yb[0, :], jnp.uint16).astype(jnp.uint32)
    hi = lax.bitcast_convert_type(yb[1, :], jnp.uint16).astype(jnp.uint32)
    out_ref[...] = lax.bitcast_convert_type(lo | (hi << 16), jnp.int32)


_proj_table = pl.pallas_call(
    _proj_body,
    grid=(VP // CBLK,),
    in_specs=[
        pl.BlockSpec((L, D), lambda i: (0, 0)),
        pl.BlockSpec((D, CBLK), lambda i: (0, i)),
    ],
    out_specs=pl.BlockSpec((CBLK,), lambda i: (i,)),
    out_shape=jax.ShapeDtypeStruct((VP,), jnp.int32),
)


def _rot(x, idx):
    return lax.gather(
        x, idx[:, None],
        lax.GatherDimensionNumbers(
            offset_dims=(), collapsed_slice_dims=(0,), start_index_map=(0,)),
        (1,), mode=lax.GatherScatterMode.PROMISE_IN_BOUNDS)


def _sc_body(tbl_hbm, sela_hbm, selb_hbm, inv_hbm, bias_hbm, out_hbm,
             tbl_v, sel_v, inv_v, bias_v, out_v, tbl_sp, sems):
    sid = lax.axis_index("s")
    wid = sid * NC + lax.axis_index("c")
    row0 = wid * RPT

    def issue(slab, buf):
        rows = pl.ds(row0 + slab * LANES, LANES)
        pltpu.make_async_copy(
            sela_hbm.at[rows], sel_v.at[buf, 0], sems[buf]).start()
        pltpu.make_async_copy(
            selb_hbm.at[rows], sel_v.at[buf, 1], sems[buf]).start()

    def drain(buf):
        rows = pl.ds(row0, LANES)
        pltpu.make_async_copy(
            sela_hbm.at[rows], sel_v.at[buf, 0], sems[buf]).wait()
        pltpu.make_async_copy(
            selb_hbm.at[rows], sel_v.at[buf, 1], sems[buf]).wait()

    issue(0, 0)
    issue(1, 1)
    pltpu.make_async_copy(
        inv_hbm.at[0, pl.ds(row0, RPT)], inv_v, sems[2]).start()
    pltpu.make_async_copy(bias_hbm, bias_v, sems[2]).start()

    @pl.when(sid == 0)
    def _():
        pltpu.sync_copy(tbl_hbm, tbl_sp)
    plsc.subcore_barrier()
    pltpu.sync_copy(tbl_sp, tbl_v)

    pltpu.make_async_copy(
        inv_hbm.at[0, pl.ds(row0, RPT)], inv_v, sems[2]).wait()
    pltpu.make_async_copy(bias_hbm, bias_v, sems[2]).wait()

    lane = lax.iota(jnp.int32, LANES)
    rot8 = (lane + 8) & 15
    rot4 = (lane + 4) & 15
    rot2 = (lane + 2) & 15
    rot1 = (lane + 1) & 15
    zero = jnp.zeros((LANES,), jnp.float32)
    bias0 = _rot(bias_v[...], jnp.zeros((LANES,), jnp.int32))
    bias1 = _rot(bias_v[...], jnp.ones((LANES,), jnp.int32))

    def tree(x):
        x = x + _rot(x, rot8)
        x = x + _rot(x, rot4)
        x = x + _rot(x, rot2)
        return x + _rot(x, rot1)

    def slab_group(g, carry):
        for buf in range(2):
            slab = g * 2 + buf
            drain(buf)

            def row_body(r2, regs):
                sr0, sr1 = regs
                for k in range(2):
                    r = r2 * 2 + k
                    a = [zero, zero, zero, zero]
                    b = [zero, zero, zero, zero]
                    for c in range(SP // LANES):
                        half, cc = (0, c) if c < 8 else (1, c - 8)
                        sel16 = sel_v[buf, half, r, pl.ds(cc * LANES, LANES)]
                        g16 = plsc.load_gather(tbl_v, [sel16])
                        pair = plsc.bitcast(g16, jnp.bfloat16)
                        u0, u1 = plsc.unpack(
                            pair, format=plsc.PackFormat.INTERLEAVED)
                        a[c % 4] = a[c % 4] + u0
                        b[c % 4] = b[c % 4] + u1
                    t0 = tree((a[0] + a[1]) + (a[2] + a[3]))
                    t1 = tree((b[0] + b[1]) + (b[2] + b[3]))
                    sr0 = jnp.where(lane == r, t0, sr0)
                    sr1 = jnp.where(lane == r, t1, sr1)
                return (sr0, sr1)
            sr0, sr1 = lax.fori_loop(0, LANES // 2, row_body, (zero, zero))

            @pl.when(slab + 2 < NSLAB)
            def _():
                issue(slab + 2, buf)

            sl = pl.ds(slab * LANES, LANES)
            iv = inv_v[sl]
            out_v[0, sl] = sr0 * iv + bias0
            out_v[1, sl] = sr1 * iv + bias1
        return carry
    lax.fori_loop(0, NSLAB // 2, slab_group, 0)

    pltpu.sync_copy(out_v.at[0], out_hbm.at[0, pl.ds(row0, RPT)])
    pltpu.sync_copy(out_v.at[1], out_hbm.at[1, pl.ds(row0, RPT)])


_sc_logits = functools.partial(
    pl.kernel,
    out_type=jax.ShapeDtypeStruct((L, B), jnp.float32),
    mesh=plsc.VectorSubcoreMesh(core_axis_name="c", subcore_axis_name="s"),
    compiler_params=pltpu.CompilerParams(
        use_tc_tiling_on_sc=False, needs_layout_passes=False),
    scratch_types=[
        pltpu.VMEM((VP,), jnp.int32),
        pltpu.VMEM((2, 2, LANES, 128), jnp.int32),
        pltpu.VMEM((RPT,), jnp.float32),
        pltpu.VMEM((LANES,), jnp.float32),
        pltpu.VMEM((L, RPT), jnp.float32),
        pltpu.VMEM_SHARED((VP,), jnp.int32),
        [pltpu.SemaphoreType.DMA] * 3,
    ],
)(_sc_body)


def kernel(input_ids, attention_mask, emb_weight, fc_w, fc_b):
    sela, selb, inv8 = _prep(input_ids.T.astype(jnp.int32),
                             attention_mask.T.astype(jnp.int32))
    tbl = _proj_table(fc_w.astype(jnp.float32), emb_weight.T)
    bias16 = jnp.zeros((LANES,), jnp.float32).at[:L].set(fc_b)
    out2 = _sc_logits(tbl, sela, selb, inv8, bias16)
    return out2.T
